# baseline (speedup 1.0000x reference)
import functools

import jax
import jax.numpy as jnp
import numpy as np
from jax import lax
from jax.experimental import pallas as pl
from jax.experimental.pallas import tpu as pltpu
from jax.experimental.pallas import tpu_sc as plsc

LEVELS = 16
TABLE_SIZE = 16384
FEATURES = 2
MINRES = (16, 16)
MAXRES = (512, 512)
_PRIME_Y = 2654435761

L = 16
P_CHUNK = 4096


def _level_resolutions(levels, minres, maxres):
    minres = np.asarray(minres, dtype=np.float64)
    maxres = np.asarray(maxres, dtype=np.float64)
    if levels > 1:
        b = np.exp((np.log(maxres) - np.log(minres)) / (levels - 1))
    else:
        b = np.ones_like(minres)
    return [tuple(int(np.round(minres[d] * (b[d] ** l)))
                  for d in range(minres.shape[0]))
            for l in range(levels)]


def _build_params():
    res = _level_resolutions(LEVELS, MINRES, MAXRES)
    return np.asarray([[r[0] - 1, r[1] - 1] for r in res],
                      dtype=np.float32).reshape(-1)


_PARAMS = _build_params()


def _make_sc_call(n_points, n_lo, n_size):
    n_half = n_size // 2
    n_chunks = n_half // P_CHUNK
    assert n_chunks * P_CHUNK == n_half and n_chunks % 2 == 0

    mesh = plsc.VectorSubcoreMesh(core_axis_name="c", subcore_axis_name="s")

    @functools.partial(
        pl.kernel,
        mesh=mesh,
        compiler_params=pltpu.CompilerParams(
            needs_layout_passes=False,
            use_tc_tiling_on_sc=False,
        ),
        out_type=jax.ShapeDtypeStruct((2 * LEVELS, n_size), jnp.float32),
        scratch_types=[
            pltpu.VMEM((2 * LEVELS,), jnp.float32),
            pltpu.VMEM((MINRES[0] * MINRES[1],), jnp.float32),
            pltpu.VMEM((MINRES[0] * MINRES[1],), jnp.float32),
            pltpu.VMEM((TABLE_SIZE,), jnp.float32),
            pltpu.VMEM((TABLE_SIZE,), jnp.float32),
        ] + [pltpu.VMEM((P_CHUNK,), jnp.float32)] * 8
          + [pltpu.SemaphoreType.DMA] * 4,
    )
    def sc_forward(coords_hbm, params_hbm, dense_hbm, hash_hbm, out_hbm,
                   params_v, d0_v, d1_v, t0_v, t1_v,
                   xb0, yb0, f0b0, f1b0, xb1, yb1, f0b1, f1b1,
                   in_sem0, in_sem1, out_sem0, out_sem1):
        s_idx = lax.axis_index("s")
        c_idx = lax.axis_index("c")
        base = n_lo + c_idx * n_half

        bufs = ((xb0, yb0, f0b0, f1b0, in_sem0, out_sem0),
                (xb1, yb1, f0b1, f1b1, in_sem1, out_sem1))

        def in_start(k, b):
            pt0 = base + k * P_CHUNK
            pltpu.async_copy(coords_hbm.at[0, pl.ds(pt0, P_CHUNK)], b[0], b[4])
            pltpu.async_copy(coords_hbm.at[1, pl.ds(pt0, P_CHUNK)], b[1], b[4])

        def in_wait(k, b):
            pt0 = base + k * P_CHUNK
            pltpu.make_async_copy(
                coords_hbm.at[0, pl.ds(pt0, P_CHUNK)], b[0], b[4]).wait()
            pltpu.make_async_copy(
                coords_hbm.at[1, pl.ds(pt0, P_CHUNK)], b[1], b[4]).wait()

        def out_start(k, b):
            pt0 = base - n_lo + k * P_CHUNK
            pltpu.async_copy(b[2], out_hbm.at[2 * s_idx, pl.ds(pt0, P_CHUNK)], b[5])
            pltpu.async_copy(b[3], out_hbm.at[2 * s_idx + 1, pl.ds(pt0, P_CHUNK)], b[5])

        def out_wait(k, b):
            pt0 = base - n_lo + k * P_CHUNK
            pltpu.make_async_copy(
                b[2], out_hbm.at[2 * s_idx, pl.ds(pt0, P_CHUNK)], b[5]).wait()
            pltpu.make_async_copy(
                b[3], out_hbm.at[2 * s_idx + 1, pl.ds(pt0, P_CHUNK)], b[5]).wait()

        in_start(0, bufs[0])
        in_start(1, bufs[1])

        pltpu.sync_copy(params_hbm, params_v)

        @pl.when(s_idx == 0)
        def _():
            pltpu.sync_copy(dense_hbm.at[0], d0_v)
            pltpu.sync_copy(dense_hbm.at[1], d1_v)

        @pl.when(s_idx > 0)
        def _():
            pltpu.sync_copy(hash_hbm.at[s_idx - 1, 0], t0_v)
            pltpu.sync_copy(hash_hbm.at[s_idx - 1, 1], t1_v)

        splat_s2 = jnp.full((L,), 2 * s_idx, jnp.int32)
        sxv = plsc.load_gather(params_v, [splat_s2])
        syv = plsc.load_gather(params_v, [splat_s2 + 1])
        rxm1 = sxv.astype(jnp.int32)
        rym1 = syv.astype(jnp.int32)

        def run(dense):
            p0_v = d0_v if dense else t0_v
            p1_v = d1_v if dense else t1_v

            def compute_chunk(b):
                xb, yb, f0b, f1b = b[0], b[1], b[2], b[3]

                @plsc.parallel_loop(0, P_CHUNK, step=L, unroll=2)
                def grp(off_i):
                    off = pl.multiple_of(off_i, L)
                    x = xb[pl.ds(off, L)] * sxv
                    y = yb[pl.ds(off, L)] * syv
                    ix0 = x.astype(jnp.int32)
                    iy0 = y.astype(jnp.int32)
                    fx = x - ix0.astype(jnp.float32)
                    fy = y - iy0.astype(jnp.float32)
                    if dense:
                        ix1 = jnp.minimum(ix0 + 1, rxm1)
                        iy1 = jnp.minimum(iy0 + 1, rym1)
                        h00 = ix0 * MINRES[1] + iy0
                        h01 = ix0 * MINRES[1] + iy1
                        h10 = ix1 * MINRES[1] + iy0
                        h11 = ix1 * MINRES[1] + iy1
                    else:
                        ix1 = ix0 + 1
                        iy1 = iy0 + 1
                        prime = jnp.uint32(_PRIME_Y)
                        mask = jnp.uint32(TABLE_SIZE - 1)
                        xu0 = ix0.astype(jnp.uint32)
                        xu1 = ix1.astype(jnp.uint32)
                        yu0 = iy0.astype(jnp.uint32) * prime
                        yu1 = iy1.astype(jnp.uint32) * prime
                        h00 = ((xu0 ^ yu0) & mask).astype(jnp.int32)
                        h01 = ((xu0 ^ yu1) & mask).astype(jnp.int32)
                        h10 = ((xu1 ^ yu0) & mask).astype(jnp.int32)
                        h11 = ((xu1 ^ yu1) & mask).astype(jnp.int32)
                    v00a = plsc.load_gather(p0_v, [h00])
                    v01a = plsc.load_gather(p0_v, [h01])
                    v10a = plsc.load_gather(p0_v, [h10])
                    v11a = plsc.load_gather(p0_v, [h11])
                    v00b = plsc.load_gather(p1_v, [h00])
                    v01b = plsc.load_gather(p1_v, [h01])
                    v10b = plsc.load_gather(p1_v, [h10])
                    v11b = plsc.load_gather(p1_v, [h11])
                    va0 = v00a + fy * (v01a - v00a)
                    va1 = v10a + fy * (v11a - v10a)
                    vb0 = v00b + fy * (v01b - v00b)
                    vb1 = v10b + fy * (v11b - v10b)
                    f0b[pl.ds(off, L)] = va0 + fx * (va1 - va0)
                    f1b[pl.ds(off, L)] = vb0 + fx * (vb1 - vb0)

            n_pairs = n_chunks // 2

            def pair_body(k2, carry):
                k = k2 * 2
                for half in (0, 1):
                    b = bufs[half]
                    kk = k + half
                    in_wait(kk, b)

                    @pl.when(k2 > 0)
                    def _():
                        out_wait(kk, b)

                    compute_chunk(b)
                    out_start(kk, b)

                    @pl.when(k2 < n_pairs - 1)
                    def _():
                        in_start(kk + 2, b)
                return carry

            lax.fori_loop(0, n_pairs, pair_body, 0)
            out_wait(0, bufs[0])
            out_wait(0, bufs[1])

        @pl.when(s_idx == 0)
        def _():
            run(True)

        @pl.when(s_idx > 0)
        def _():
            run(False)

    return sc_forward


def kernel(coords, dense_table, hash_tables):
    n_points = coords.shape[1]
    half = n_points // 2
    sc_a = _make_sc_call(n_points, 0, half)
    sc_b = _make_sc_call(n_points, half, half)
    params = jnp.asarray(_PARAMS)
    dense_planes = jnp.moveaxis(dense_table.reshape(-1, FEATURES), -1, 0)
    hash_planes = jnp.moveaxis(hash_tables, -1, 1)
    out_a = sc_a(coords, params, dense_planes, hash_planes)
    out_b = sc_b(coords, params, dense_planes, hash_planes)
    return jnp.concatenate([out_a.T, out_b.T], axis=0)


# speedup vs baseline: 1.1348x; 1.1348x over previous
import functools

import jax
import jax.numpy as jnp
import numpy as np
from jax import lax
from jax.experimental import pallas as pl
from jax.experimental.pallas import tpu as pltpu
from jax.experimental.pallas import tpu_sc as plsc

LEVELS = 16
TABLE_SIZE = 16384
FEATURES = 2
MINRES = (16, 16)
MAXRES = (512, 512)
_PRIME_Y = 2654435761

L = 16
P_CHUNK = 4096


def _level_resolutions(levels, minres, maxres):
    minres = np.asarray(minres, dtype=np.float64)
    maxres = np.asarray(maxres, dtype=np.float64)
    if levels > 1:
        b = np.exp((np.log(maxres) - np.log(minres)) / (levels - 1))
    else:
        b = np.ones_like(minres)
    return [tuple(int(np.round(minres[d] * (b[d] ** l)))
                  for d in range(minres.shape[0]))
            for l in range(levels)]


def _build_params():
    res = _level_resolutions(LEVELS, MINRES, MAXRES)
    return np.asarray([[r[0] - 1, r[1] - 1] for r in res],
                      dtype=np.float32).reshape(-1)


_PARAMS = _build_params()


def _make_sc_call(n_points):
    n_half = n_points // 2
    n_chunks = n_half // P_CHUNK
    assert n_chunks * P_CHUNK == n_half and n_chunks % 2 == 0

    mesh = plsc.VectorSubcoreMesh(core_axis_name="c", subcore_axis_name="s")

    @functools.partial(
        pl.kernel,
        mesh=mesh,
        compiler_params=pltpu.CompilerParams(
            needs_layout_passes=False,
            use_tc_tiling_on_sc=False,
        ),
        out_type=jax.ShapeDtypeStruct((2 * LEVELS, n_points), jnp.float32),
        scratch_types=[
            pltpu.VMEM((2 * LEVELS,), jnp.float32),
            pltpu.VMEM((MINRES[0] * MINRES[1],), jnp.float32),
            pltpu.VMEM((MINRES[0] * MINRES[1],), jnp.float32),
            pltpu.VMEM((TABLE_SIZE,), jnp.float32),
            pltpu.VMEM((TABLE_SIZE,), jnp.float32),
        ] + [pltpu.VMEM((P_CHUNK,), jnp.float32)] * 8
          + [pltpu.SemaphoreType.DMA] * 4,
    )
    def sc_forward(coords_hbm, params_hbm, dense_hbm, hash_hbm, out_hbm,
                   params_v, d0_v, d1_v, t0_v, t1_v,
                   xb0, yb0, f0b0, f1b0, xb1, yb1, f0b1, f1b1,
                   in_sem0, in_sem1, out_sem0, out_sem1):
        s_idx = lax.axis_index("s")
        c_idx = lax.axis_index("c")
        base = c_idx * n_half

        bufs = ((xb0, yb0, f0b0, f1b0, in_sem0, out_sem0),
                (xb1, yb1, f0b1, f1b1, in_sem1, out_sem1))

        def in_start(k, b):
            pt0 = base + k * P_CHUNK
            pltpu.async_copy(coords_hbm.at[0, pl.ds(pt0, P_CHUNK)], b[0], b[4])
            pltpu.async_copy(coords_hbm.at[1, pl.ds(pt0, P_CHUNK)], b[1], b[4])

        def in_wait(k, b):
            pt0 = base + k * P_CHUNK
            pltpu.make_async_copy(
                coords_hbm.at[0, pl.ds(pt0, P_CHUNK)], b[0], b[4]).wait()
            pltpu.make_async_copy(
                coords_hbm.at[1, pl.ds(pt0, P_CHUNK)], b[1], b[4]).wait()

        def out_start(k, b):
            pt0 = base + k * P_CHUNK
            pltpu.async_copy(b[2], out_hbm.at[2 * s_idx, pl.ds(pt0, P_CHUNK)], b[5])
            pltpu.async_copy(b[3], out_hbm.at[2 * s_idx + 1, pl.ds(pt0, P_CHUNK)], b[5])

        def out_wait(k, b):
            pt0 = base + k * P_CHUNK
            pltpu.make_async_copy(
                b[2], out_hbm.at[2 * s_idx, pl.ds(pt0, P_CHUNK)], b[5]).wait()
            pltpu.make_async_copy(
                b[3], out_hbm.at[2 * s_idx + 1, pl.ds(pt0, P_CHUNK)], b[5]).wait()

        in_start(0, bufs[0])
        in_start(1, bufs[1])

        pltpu.sync_copy(params_hbm, params_v)

        @pl.when(s_idx == 0)
        def _():
            pltpu.sync_copy(dense_hbm.at[0], d0_v)
            pltpu.sync_copy(dense_hbm.at[1], d1_v)

        @pl.when(s_idx > 0)
        def _():
            pltpu.sync_copy(hash_hbm.at[s_idx - 1, 0], t0_v)
            pltpu.sync_copy(hash_hbm.at[s_idx - 1, 1], t1_v)

        splat_s2 = jnp.full((L,), 2 * s_idx, jnp.int32)
        sxv = plsc.load_gather(params_v, [splat_s2])
        syv = plsc.load_gather(params_v, [splat_s2 + 1])
        rxm1 = sxv.astype(jnp.int32)
        rym1 = syv.astype(jnp.int32)

        def run(dense):
            p0_v = d0_v if dense else t0_v
            p1_v = d1_v if dense else t1_v

            def compute_chunk(b):
                xb, yb, f0b, f1b = b[0], b[1], b[2], b[3]

                @plsc.parallel_loop(0, P_CHUNK, step=L, unroll=2)
                def grp(off_i):
                    off = pl.multiple_of(off_i, L)
                    x = xb[pl.ds(off, L)] * sxv
                    y = yb[pl.ds(off, L)] * syv
                    ix0 = x.astype(jnp.int32)
                    iy0 = y.astype(jnp.int32)
                    fx = x - ix0.astype(jnp.float32)
                    fy = y - iy0.astype(jnp.float32)
                    if dense:
                        ix1 = jnp.minimum(ix0 + 1, rxm1)
                        iy1 = jnp.minimum(iy0 + 1, rym1)
                        h00 = ix0 * MINRES[1] + iy0
                        h01 = ix0 * MINRES[1] + iy1
                        h10 = ix1 * MINRES[1] + iy0
                        h11 = ix1 * MINRES[1] + iy1
                    else:
                        ix1 = ix0 + 1
                        iy1 = iy0 + 1
                        prime = jnp.uint32(_PRIME_Y)
                        mask = jnp.uint32(TABLE_SIZE - 1)
                        xu0 = ix0.astype(jnp.uint32)
                        xu1 = ix1.astype(jnp.uint32)
                        yu0 = iy0.astype(jnp.uint32) * prime
                        yu1 = iy1.astype(jnp.uint32) * prime
                        h00 = ((xu0 ^ yu0) & mask).astype(jnp.int32)
                        h01 = ((xu0 ^ yu1) & mask).astype(jnp.int32)
                        h10 = ((xu1 ^ yu0) & mask).astype(jnp.int32)
                        h11 = ((xu1 ^ yu1) & mask).astype(jnp.int32)
                    v00a = plsc.load_gather(p0_v, [h00])
                    v01a = plsc.load_gather(p0_v, [h01])
                    v10a = plsc.load_gather(p0_v, [h10])
                    v11a = plsc.load_gather(p0_v, [h11])
                    v00b = plsc.load_gather(p1_v, [h00])
                    v01b = plsc.load_gather(p1_v, [h01])
                    v10b = plsc.load_gather(p1_v, [h10])
                    v11b = plsc.load_gather(p1_v, [h11])
                    va0 = v00a + fy * (v01a - v00a)
                    va1 = v10a + fy * (v11a - v10a)
                    vb0 = v00b + fy * (v01b - v00b)
                    vb1 = v10b + fy * (v11b - v10b)
                    f0b[pl.ds(off, L)] = va0 + fx * (va1 - va0)
                    f1b[pl.ds(off, L)] = vb0 + fx * (vb1 - vb0)

            n_pairs = n_chunks // 2

            def pair_body(k2, carry):
                k = k2 * 2
                for half in (0, 1):
                    b = bufs[half]
                    kk = k + half
                    in_wait(kk, b)

                    @pl.when(k2 > 0)
                    def _():
                        out_wait(kk, b)

                    compute_chunk(b)
                    out_start(kk, b)

                    @pl.when(k2 < n_pairs - 1)
                    def _():
                        in_start(kk + 2, b)
                return carry

            lax.fori_loop(0, n_pairs, pair_body, 0)
            out_wait(0, bufs[0])
            out_wait(0, bufs[1])

        @pl.when(s_idx == 0)
        def _():
            run(True)

        @pl.when(s_idx > 0)
        def _():
            run(False)

    return sc_forward


def kernel(coords, dense_table, hash_tables):
    n_points = coords.shape[1]
    sc_forward = _make_sc_call(n_points)
    params = jnp.asarray(_PARAMS)
    dense_planes = jnp.moveaxis(dense_table.reshape(-1, FEATURES), -1, 0)
    hash_planes = jnp.moveaxis(hash_tables, -1, 1)
    out32 = sc_forward(coords, params, dense_planes, hash_planes)
    return out32.T
